# baseline (speedup 1.0000x reference)
import functools

import jax
import jax.numpy as jnp
from jax import lax
from jax.experimental import pallas as pl
from jax.experimental.pallas import tpu as pltpu
from jax.experimental.pallas import tpu_sc as plsc


def _sc_dot_lookup(B, V, D):
    info = plsc.get_sparse_core_info()
    NC, NS, L = info.num_cores, info.num_subcores, info.num_lanes
    NW = NC * NS
    assert B % NW == 0
    b_per_w = B // NW
    n_chunks = 4
    chunk = b_per_w // n_chunks
    W = 2 * D
    WR = D // 2
    V4 = V // 4

    mesh = plsc.VectorSubcoreMesh(core_axis_name="c", subcore_axis_name="s")

    @functools.partial(
        pl.kernel,
        mesh=mesh,
        out_type=jax.ShapeDtypeStruct((B,), jnp.float32),
        compiler_params=pltpu.CompilerParams(needs_layout_passes=False),
        scratch_types=[
            pltpu.VMEM((n_chunks, chunk), jnp.int32),
            pltpu.VMEM((n_chunks, chunk), jnp.int32),
            pltpu.VMEM((n_chunks, chunk), jnp.int32),
            pltpu.VMEM((n_chunks, chunk), jnp.int32),
            pltpu.VMEM((chunk, W), jnp.int32),
            pltpu.VMEM((chunk, W), jnp.int32),
            pltpu.VMEM((chunk, W), jnp.int32),
            pltpu.VMEM((chunk, W), jnp.int32),
            pltpu.VMEM((b_per_w,), jnp.float32),
            pltpu.SemaphoreType.DMA,
            pltpu.SemaphoreType.DMA,
            pltpu.SemaphoreType.DMA,
            pltpu.SemaphoreType.DMA,
        ],
    )
    def k(tgt_hbm, ctx_hbm, tab_hbm, out_hbm,
          idx_t, idx_c, sidx_t, sidx_c, rt0, rt1, rc0, rc1, out_v,
          sem_t0, sem_t1, sem_c0, sem_c1):
        wid = lax.axis_index("s") * NC + lax.axis_index("c")
        base = wid * b_per_w
        rows_t = (rt0, rt1)
        rows_c = (rc0, rc1)
        sems_t = (sem_t0, sem_t1)
        sems_c = (sem_c0, sem_c1)

        for j in range(n_chunks):
            pltpu.sync_copy(tgt_hbm.at[pl.ds(base + j * chunk, chunk)], idx_t.at[j])
            pltpu.sync_copy(ctx_hbm.at[pl.ds(base + j * chunk, chunk)], idx_c.at[j])

        for j in range(n_chunks):
            for g in range(chunk // L):
                sl = pl.ds(g * L, L)
                sidx_t[j, sl] = lax.shift_right_logical(idx_t[j, sl], 2)
                sidx_c[j, sl] = lax.shift_right_logical(idx_c[j, sl], 2) + V4

        def fire(j):
            b = j % 2
            ct = pltpu.async_copy(tab_hbm.at[sidx_t.at[j]], rows_t[b], sems_t[b])
            cc = pltpu.async_copy(tab_hbm.at[sidx_c.at[j]], rows_c[b], sems_c[b])
            return ct, cc

        lanes = lax.iota(jnp.int32, L)
        inflight = [fire(0), fire(1)]

        for j in range(n_chunks):
            b = j % 2
            ct, cc = inflight[b]
            ct.wait()
            cc.wait()
            rt = rows_t[b]
            rc = rows_c[b]

            def group_body(g, carry, j=j, rt=rt, rc=rc):
                sl = pl.ds(g * L, L)
                row = g * L + lanes
                qt = jnp.bitwise_and(idx_t[j, sl], 3) * WR
                qc = jnp.bitwise_and(idx_c[j, sl], 3) * WR
                acc = jnp.zeros((L,), jnp.float32)
                for w in range(WR):
                    diag = jnp.bitwise_and(lanes + w, WR - 1)
                    tw = plsc.load_gather(rt, [row, qt + diag])
                    cw = plsc.load_gather(rc, [row, qc + diag])
                    ta, tb = plsc.unpack(plsc.bitcast(tw, jnp.bfloat16),
                                         format=plsc.PackFormat.INTERLEAVED)
                    ca, cb = plsc.unpack(plsc.bitcast(cw, jnp.bfloat16),
                                         format=plsc.PackFormat.INTERLEAVED)
                    acc = acc + ta * ca + tb * cb
                out_v[pl.ds(j * chunk + g * L, L)] = acc
                return carry

            lax.fori_loop(0, chunk // L, group_body, 0)

            if j + 2 < n_chunks:
                inflight[b] = fire(j + 2)

        pltpu.sync_copy(out_v, out_hbm.at[pl.ds(base, b_per_w)])

    return k


def kernel(target, context, target_table, context_table):
    B = target.shape[0]
    V, D = target_table.shape

    def pack(tab):
        t16 = tab.astype(jnp.bfloat16).reshape(V // 4, 2 * D, 2)
        return lax.bitcast_convert_type(t16, jnp.int32)

    tab = jnp.concatenate([pack(target_table), pack(context_table)], axis=0)
    k = _sc_dot_lookup(B, V, D)
    return k(target.astype(jnp.int32), context.astype(jnp.int32), tab)


# speedup vs baseline: 52.2092x; 52.2092x over previous
import functools

import jax
import jax.numpy as jnp
from jax import lax
from jax.experimental import pallas as pl
from jax.experimental.pallas import tpu as pltpu
from jax.experimental.pallas import tpu_sc as plsc


_VB = 2048


def _pack_block(xT_ref, out_ref):
    t = lax.transpose(xT_ref[...], (1, 0))
    bits = lax.bitcast_convert_type(t, jnp.uint32)
    c16 = jnp.uint32(16)
    r = lax.shift_right_logical(
        bits + jnp.uint32(0x7FFF)
        + jnp.bitwise_and(lax.shift_right_logical(bits, c16), jnp.uint32(1)),
        c16)
    pa = jnp.bitwise_or(r, lax.shift_left(pltpu.roll(r, _VB // 2, 0), c16))
    h0 = lax.slice(pa, (0, 0), (_VB // 4, 64))
    h1 = lax.slice(pa, (_VB // 4, 0), (_VB // 2, 64))
    out_ref[...] = lax.bitcast_convert_type(
        jnp.concatenate([h0, h1], axis=1), jnp.int32)


def _tc_pack(V, D):
    grid = pl.cdiv(V, _VB)
    return pl.pallas_call(
        _pack_block,
        grid=(grid,),
        in_specs=[pl.BlockSpec((D, _VB), lambda b: (0, b))],
        out_specs=pl.BlockSpec((_VB // 4, 2 * D), lambda b: (b, 0)),
        out_shape=jax.ShapeDtypeStruct((grid * (_VB // 4), 2 * D), jnp.int32),
    )


def _sc_dot_lookup(B, V, D):
    info = plsc.get_sparse_core_info()
    NC, NS, L = info.num_cores, info.num_subcores, info.num_lanes
    NW = NC * NS
    assert B % NW == 0
    b_per_w = B // NW
    n_chunks = 4
    chunk = b_per_w // n_chunks
    W = 2 * D
    V4 = V // 4

    mesh = plsc.VectorSubcoreMesh(core_axis_name="c", subcore_axis_name="s")

    @functools.partial(
        pl.kernel,
        mesh=mesh,
        out_type=jax.ShapeDtypeStruct((B,), jnp.float32),
        compiler_params=pltpu.CompilerParams(needs_layout_passes=False),
        scratch_types=[
            pltpu.VMEM((n_chunks, chunk), jnp.int32),
            pltpu.VMEM((n_chunks, chunk), jnp.int32),
            pltpu.VMEM((n_chunks, chunk), jnp.int32),
            pltpu.VMEM((n_chunks, chunk), jnp.int32),
            pltpu.VMEM((chunk, W), jnp.int32),
            pltpu.VMEM((chunk, W), jnp.int32),
            pltpu.VMEM((chunk, W), jnp.int32),
            pltpu.VMEM((chunk, W), jnp.int32),
            pltpu.VMEM((b_per_w,), jnp.float32),
            pltpu.SemaphoreType.DMA,
            pltpu.SemaphoreType.DMA,
            pltpu.SemaphoreType.DMA,
            pltpu.SemaphoreType.DMA,
        ],
    )
    def k(tgt_hbm, ctx_hbm, ttab_hbm, ctab_hbm, out_hbm,
          idx_t, idx_c, sidx_t, sidx_c, rt0, rt1, rc0, rc1, out_v,
          sem_t0, sem_t1, sem_c0, sem_c1):
        wid = lax.axis_index("s") * NC + lax.axis_index("c")
        base = wid * b_per_w
        rows_t = (rt0, rt1)
        rows_c = (rc0, rc1)
        sems_t = (sem_t0, sem_t1)
        sems_c = (sem_c0, sem_c1)

        for j in range(n_chunks):
            pltpu.sync_copy(tgt_hbm.at[pl.ds(base + j * chunk, chunk)], idx_t.at[j])
            pltpu.sync_copy(ctx_hbm.at[pl.ds(base + j * chunk, chunk)], idx_c.at[j])

        for j in range(n_chunks):
            for g in range(chunk // L):
                sl = pl.ds(g * L, L)
                for src, dst in ((idx_t, sidx_t), (idx_c, sidx_c)):
                    v = src[j, sl]
                    dst[j, sl] = jnp.bitwise_or(
                        lax.shift_left(lax.shift_right_logical(v, 11), 9),
                        jnp.bitwise_and(v, 511))

        def fire(j):
            b = j % 2
            ct = pltpu.async_copy(ttab_hbm.at[sidx_t.at[j]], rows_t[b], sems_t[b])
            cc = pltpu.async_copy(ctab_hbm.at[sidx_c.at[j]], rows_c[b], sems_c[b])
            return ct, cc

        lanes = lax.iota(jnp.int32, L)
        inflight = [fire(0), fire(1)]

        for j in range(n_chunks):
            b = j % 2
            ct, cc = inflight[b]
            ct.wait()
            cc.wait()
            rt = rows_t[b]
            rc = rows_c[b]

            def group_body(g, carry, j=j, rt=rt, rc=rc):
                sl = pl.ds(g * L, L)
                row = g * L + lanes
                iv_t = idx_t[j, sl]
                iv_c = idx_c[j, sl]
                pt = jnp.bitwise_and(lax.shift_right_logical(iv_t, 9), 1) * D
                pc = jnp.bitwise_and(lax.shift_right_logical(iv_c, 9), 1) * D
                sh_t = (1 - jnp.bitwise_and(
                    lax.shift_right_logical(iv_t, 10), 1)) * 16
                sh_c = (1 - jnp.bitwise_and(
                    lax.shift_right_logical(iv_c, 10), 1)) * 16
                hi_mask = jnp.full((L,), -65536, jnp.int32)
                acc = jnp.zeros((L,), jnp.float32)
                for w in range(D):
                    diag = jnp.bitwise_and(lanes + w, D - 1)
                    tw = plsc.load_gather(rt, [row, pt + diag])
                    cw = plsc.load_gather(rc, [row, pc + diag])
                    tf = plsc.bitcast(
                        jnp.bitwise_and(lax.shift_left(tw, sh_t), hi_mask),
                        jnp.float32)
                    cf = plsc.bitcast(
                        jnp.bitwise_and(lax.shift_left(cw, sh_c), hi_mask),
                        jnp.float32)
                    acc = acc + tf * cf
                out_v[pl.ds(j * chunk + g * L, L)] = acc
                return carry

            lax.fori_loop(0, chunk // L, group_body, 0)

            if j + 2 < n_chunks:
                inflight[b] = fire(j + 2)

        pltpu.sync_copy(out_v, out_hbm.at[pl.ds(base, b_per_w)])

    return k


def kernel(target, context, target_table, context_table):
    B = target.shape[0]
    V, D = target_table.shape
    pack = _tc_pack(V, D)
    ttab = pack(target_table.T)
    ctab = pack(context_table.T)
    k = _sc_dot_lookup(B, V, D)
    return k(target.astype(jnp.int32), context.astype(jnp.int32), ttab, ctab)


# speedup vs baseline: 83.0251x; 1.5902x over previous
import functools

import jax
import jax.numpy as jnp
from jax import lax
from jax.experimental import pallas as pl
from jax.experimental.pallas import tpu as pltpu
from jax.experimental.pallas import tpu_sc as plsc


_VB = 4096
_HB = _VB // 2
_QB = _VB // 4
_LOG_VB = 12
_LOG_QB = 10


def _pack_block(xT_ref, out_ref):
    D = xT_ref.shape[0]
    bits = lax.bitcast_convert_type(xT_ref[...], jnp.uint32)
    half = jnp.uint32(0x8000)
    lo = lax.shift_right_logical(
        lax.slice(bits, (0, 0), (D, _HB)) + half, jnp.uint32(16))
    hi = jnp.bitwise_and(
        lax.slice(bits, (0, _HB), (D, _VB)) + half, jnp.uint32(0xFFFF0000))
    pa = jnp.bitwise_or(lo, hi)
    t = lax.transpose(pa, (1, 0))
    h0 = lax.slice(t, (0, 0), (_QB, D))
    h1 = lax.slice(t, (_QB, 0), (_HB, D))
    out_ref[...] = lax.bitcast_convert_type(
        jnp.concatenate([h0, h1], axis=1), jnp.int32)


def _tc_pack(V, D):
    grid = pl.cdiv(V, _VB)
    return pl.pallas_call(
        _pack_block,
        grid=(grid,),
        in_specs=[pl.BlockSpec((D, _VB), lambda b: (0, b))],
        out_specs=pl.BlockSpec((_QB, 2 * D), lambda b: (b, 0)),
        out_shape=jax.ShapeDtypeStruct((grid * _QB, 2 * D), jnp.int32),
    )


def _sc_dot_lookup(B, V, D):
    info = plsc.get_sparse_core_info()
    NC, NS, L = info.num_cores, info.num_subcores, info.num_lanes
    NW = NC * NS
    assert B % NW == 0
    b_per_w = B // NW
    n_chunks = 4
    chunk = b_per_w // n_chunks
    W = 2 * D
    V4 = V // 4

    mesh = plsc.VectorSubcoreMesh(core_axis_name="c", subcore_axis_name="s")

    @functools.partial(
        pl.kernel,
        mesh=mesh,
        out_type=jax.ShapeDtypeStruct((B,), jnp.float32),
        compiler_params=pltpu.CompilerParams(needs_layout_passes=False),
        scratch_types=[
            pltpu.VMEM((n_chunks, chunk), jnp.int32),
            pltpu.VMEM((n_chunks, chunk), jnp.int32),
            pltpu.VMEM((n_chunks, chunk), jnp.int32),
            pltpu.VMEM((n_chunks, chunk), jnp.int32),
            pltpu.VMEM((chunk, W), jnp.int32),
            pltpu.VMEM((chunk, W), jnp.int32),
            pltpu.VMEM((chunk, W), jnp.int32),
            pltpu.VMEM((chunk, W), jnp.int32),
            pltpu.VMEM((b_per_w,), jnp.float32),
            pltpu.SemaphoreType.DMA,
            pltpu.SemaphoreType.DMA,
            pltpu.SemaphoreType.DMA,
            pltpu.SemaphoreType.DMA,
        ],
    )
    def k(tgt_hbm, ctx_hbm, ttab_hbm, ctab_hbm, out_hbm,
          idx_t, idx_c, sidx_t, sidx_c, rt0, rt1, rc0, rc1, out_v,
          sem_t0, sem_t1, sem_c0, sem_c1):
        wid = lax.axis_index("s") * NC + lax.axis_index("c")
        base = wid * b_per_w
        rows_t = (rt0, rt1)
        rows_c = (rc0, rc1)
        sems_t = (sem_t0, sem_t1)
        sems_c = (sem_c0, sem_c1)

        for j in range(n_chunks):
            pltpu.sync_copy(tgt_hbm.at[pl.ds(base + j * chunk, chunk)], idx_t.at[j])
            pltpu.sync_copy(ctx_hbm.at[pl.ds(base + j * chunk, chunk)], idx_c.at[j])

        for j in range(n_chunks):
            for g in range(chunk // L):
                sl = pl.ds(g * L, L)
                for src, dst in ((idx_t, sidx_t), (idx_c, sidx_c)):
                    v = src[j, sl]
                    dst[j, sl] = jnp.bitwise_or(
                        lax.shift_left(
                            lax.shift_right_logical(v, _LOG_VB), _LOG_QB),
                        jnp.bitwise_and(v, _QB - 1))

        def fire(j):
            b = j % 2
            ct = pltpu.async_copy(ttab_hbm.at[sidx_t.at[j]], rows_t[b], sems_t[b])
            cc = pltpu.async_copy(ctab_hbm.at[sidx_c.at[j]], rows_c[b], sems_c[b])
            return ct, cc

        lanes = lax.iota(jnp.int32, L)
        inflight = [fire(0), fire(1)]

        for j in range(n_chunks):
            b = j % 2
            ct, cc = inflight[b]
            ct.wait()
            cc.wait()
            rt = rows_t[b]
            rc = rows_c[b]

            def group_body(g, carry, j=j, rt=rt, rc=rc):
                sl = pl.ds(g * L, L)
                row = g * L + lanes
                iv_t = idx_t[j, sl]
                iv_c = idx_c[j, sl]
                pt = jnp.bitwise_and(
                    lax.shift_right_logical(iv_t, _LOG_QB), 1) * D
                pc = jnp.bitwise_and(
                    lax.shift_right_logical(iv_c, _LOG_QB), 1) * D
                sh_t = (1 - jnp.bitwise_and(
                    lax.shift_right_logical(iv_t, _LOG_VB - 1), 1)) * 16
                sh_c = (1 - jnp.bitwise_and(
                    lax.shift_right_logical(iv_c, _LOG_VB - 1), 1)) * 16
                hi_mask = jnp.full((L,), -65536, jnp.int32)
                acc = jnp.zeros((L,), jnp.float32)
                for w in range(D):
                    diag = jnp.bitwise_and(lanes + w, D - 1)
                    tw = plsc.load_gather(rt, [row, pt + diag])
                    cw = plsc.load_gather(rc, [row, pc + diag])
                    tf = plsc.bitcast(
                        jnp.bitwise_and(lax.shift_left(tw, sh_t), hi_mask),
                        jnp.float32)
                    cf = plsc.bitcast(
                        jnp.bitwise_and(lax.shift_left(cw, sh_c), hi_mask),
                        jnp.float32)
                    acc = acc + tf * cf
                out_v[pl.ds(j * chunk + g * L, L)] = acc
                return carry

            lax.fori_loop(0, chunk // L, group_body, 0)

            if j + 2 < n_chunks:
                inflight[b] = fire(j + 2)

        pltpu.sync_copy(out_v, out_hbm.at[pl.ds(base, b_per_w)])

    return k


def kernel(target, context, target_table, context_table):
    B = target.shape[0]
    V, D = target_table.shape
    pack = _tc_pack(V, D)
    ttab = pack(target_table.T)
    ctab = pack(context_table.T)
    k = _sc_dot_lookup(B, V, D)
    return k(target.astype(jnp.int32), context.astype(jnp.int32), ttab, ctab)


# speedup vs baseline: 108.5405x; 1.3073x over previous
import functools

import jax
import jax.numpy as jnp
from jax import lax
from jax.experimental import pallas as pl
from jax.experimental.pallas import tpu as pltpu
from jax.experimental.pallas import tpu_sc as plsc


_VB = 8192
_HB = _VB // 2
_QB = _VB // 4
_LOG_VB = 13
_LOG_QB = 11


def _pack_block(xT_ref, out_ref):
    D = xT_ref.shape[0]
    bits = lax.bitcast_convert_type(xT_ref[...], jnp.uint32)
    half = jnp.uint32(0x8000)
    lo = lax.shift_right_logical(
        lax.slice(bits, (0, 0), (D, _HB)) + half, jnp.uint32(16))
    hi = jnp.bitwise_and(
        lax.slice(bits, (0, _HB), (D, _VB)) + half, jnp.uint32(0xFFFF0000))
    pa = jnp.bitwise_or(lo, hi)
    t = lax.transpose(pa, (1, 0))
    h0 = lax.slice(t, (0, 0), (_QB, D))
    h1 = lax.slice(t, (_QB, 0), (_HB, D))
    out_ref[...] = lax.bitcast_convert_type(
        jnp.concatenate([h0, h1], axis=1), jnp.int32)


def _tc_pack(V, D):
    grid = pl.cdiv(V, _VB)
    return pl.pallas_call(
        _pack_block,
        grid=(grid,),
        in_specs=[pl.BlockSpec((D, _VB), lambda b: (0, b))],
        out_specs=pl.BlockSpec((_QB, 2 * D), lambda b: (b, 0)),
        out_shape=jax.ShapeDtypeStruct((grid * _QB, 2 * D), jnp.int32),
    )


def _sc_dot_lookup(B, V, D):
    info = plsc.get_sparse_core_info()
    NC, NS, L = info.num_cores, info.num_subcores, info.num_lanes
    NW = NC * NS
    assert B % NW == 0
    b_per_w = B // NW
    n_chunks = 4
    chunk = b_per_w // n_chunks
    W = 2 * D
    V4 = V // 4

    mesh = plsc.VectorSubcoreMesh(core_axis_name="c", subcore_axis_name="s")

    @functools.partial(
        pl.kernel,
        mesh=mesh,
        out_type=jax.ShapeDtypeStruct((B,), jnp.float32),
        compiler_params=pltpu.CompilerParams(needs_layout_passes=False),
        scratch_types=[
            pltpu.VMEM((n_chunks, chunk), jnp.int32),
            pltpu.VMEM((n_chunks, chunk), jnp.int32),
            pltpu.VMEM((n_chunks, chunk), jnp.int32),
            pltpu.VMEM((n_chunks, chunk), jnp.int32),
            pltpu.VMEM((chunk, W), jnp.int32),
            pltpu.VMEM((chunk, W), jnp.int32),
            pltpu.VMEM((chunk, W), jnp.int32),
            pltpu.VMEM((chunk, W), jnp.int32),
            pltpu.VMEM((b_per_w,), jnp.float32),
            pltpu.SemaphoreType.DMA,
            pltpu.SemaphoreType.DMA,
            pltpu.SemaphoreType.DMA,
            pltpu.SemaphoreType.DMA,
        ],
    )
    def k(tgt_hbm, ctx_hbm, ttab_hbm, ctab_hbm, out_hbm,
          idx_t, idx_c, sidx_t, sidx_c, rt0, rt1, rc0, rc1, out_v,
          sem_t0, sem_t1, sem_c0, sem_c1):
        wid = lax.axis_index("s") * NC + lax.axis_index("c")
        base = wid * b_per_w
        rows_t = (rt0, rt1)
        rows_c = (rc0, rc1)
        sems_t = (sem_t0, sem_t1)
        sems_c = (sem_c0, sem_c1)

        for j in range(n_chunks):
            pltpu.sync_copy(tgt_hbm.at[pl.ds(base + j * chunk, chunk)], idx_t.at[j])
            pltpu.sync_copy(ctx_hbm.at[pl.ds(base + j * chunk, chunk)], idx_c.at[j])

        for j in range(n_chunks):
            for g in range(chunk // L):
                sl = pl.ds(g * L, L)
                for src, dst in ((idx_t, sidx_t), (idx_c, sidx_c)):
                    v = src[j, sl]
                    dst[j, sl] = jnp.bitwise_or(
                        lax.shift_left(
                            lax.shift_right_logical(v, _LOG_VB), _LOG_QB),
                        jnp.bitwise_and(v, _QB - 1))

        def fire(j):
            b = j % 2
            ct = pltpu.async_copy(ttab_hbm.at[sidx_t.at[j]], rows_t[b], sems_t[b])
            cc = pltpu.async_copy(ctab_hbm.at[sidx_c.at[j]], rows_c[b], sems_c[b])
            return ct, cc

        lanes = lax.iota(jnp.int32, L)
        inflight = [fire(0), fire(1)]

        for j in range(n_chunks):
            b = j % 2
            ct, cc = inflight[b]
            ct.wait()
            cc.wait()
            rt = rows_t[b]
            rc = rows_c[b]

            def group_body(g, carry, j=j, rt=rt, rc=rc):
                sl = pl.ds(g * L, L)
                row = g * L + lanes
                iv_t = idx_t[j, sl]
                iv_c = idx_c[j, sl]
                pt = jnp.bitwise_and(
                    lax.shift_right_logical(iv_t, _LOG_QB), 1) * D
                pc = jnp.bitwise_and(
                    lax.shift_right_logical(iv_c, _LOG_QB), 1) * D
                sh_t = (1 - jnp.bitwise_and(
                    lax.shift_right_logical(iv_t, _LOG_VB - 1), 1)) * 16
                sh_c = (1 - jnp.bitwise_and(
                    lax.shift_right_logical(iv_c, _LOG_VB - 1), 1)) * 16
                hi_mask = jnp.full((L,), -65536, jnp.int32)
                acc = jnp.zeros((L,), jnp.float32)
                for w in range(D):
                    diag = jnp.bitwise_and(lanes + w, D - 1)
                    tw = plsc.load_gather(rt, [row, pt + diag])
                    cw = plsc.load_gather(rc, [row, pc + diag])
                    tf = plsc.bitcast(
                        jnp.bitwise_and(lax.shift_left(tw, sh_t), hi_mask),
                        jnp.float32)
                    cf = plsc.bitcast(
                        jnp.bitwise_and(lax.shift_left(cw, sh_c), hi_mask),
                        jnp.float32)
                    acc = acc + tf * cf
                out_v[pl.ds(j * chunk + g * L, L)] = acc
                return carry

            lax.fori_loop(0, chunk // L, group_body, 0)

            if j + 2 < n_chunks:
                inflight[b] = fire(j + 2)

        pltpu.sync_copy(out_v, out_hbm.at[pl.ds(base, b_per_w)])

    return k


def kernel(target, context, target_table, context_table):
    B = target.shape[0]
    V, D = target_table.shape
    pack = _tc_pack(V, D)
    ttab = pack(target_table.T)
    ctab = pack(context_table.T)
    k = _sc_dot_lookup(B, V, D)
    return k(target.astype(jnp.int32), context.astype(jnp.int32), ttab, ctab)


# speedup vs baseline: 132.5711x; 1.2214x over previous
import functools

import jax
import jax.numpy as jnp
from jax import lax
from jax.experimental import pallas as pl
from jax.experimental.pallas import tpu as pltpu
from jax.experimental.pallas import tpu_sc as plsc


_VB = 16384
_HB = _VB // 2
_QB = _VB // 4
_LOG_VB = 14
_LOG_QB = 12


def _pack_block(xT_ref, out_ref):
    D = xT_ref.shape[0]
    bits = lax.bitcast_convert_type(xT_ref[...], jnp.uint32)
    half = jnp.uint32(0x8000)
    lo = lax.shift_right_logical(
        lax.slice(bits, (0, 0), (D, _HB)) + half, jnp.uint32(16))
    hi = jnp.bitwise_and(
        lax.slice(bits, (0, _HB), (D, _VB)) + half, jnp.uint32(0xFFFF0000))
    pa = jnp.bitwise_or(lo, hi)
    t = lax.transpose(pa, (1, 0))
    h0 = lax.slice(t, (0, 0), (_QB, D))
    h1 = lax.slice(t, (_QB, 0), (_HB, D))
    out_ref[...] = lax.bitcast_convert_type(
        jnp.concatenate([h0, h1], axis=1), jnp.int32)


def _tc_pack(V, D):
    grid = pl.cdiv(V, _VB)
    return pl.pallas_call(
        _pack_block,
        grid=(grid,),
        in_specs=[pl.BlockSpec((D, _VB), lambda b: (0, b))],
        out_specs=pl.BlockSpec((_QB, 2 * D), lambda b: (b, 0)),
        out_shape=jax.ShapeDtypeStruct((grid * _QB, 2 * D), jnp.int32),
    )


def _sc_dot_lookup(B, V, D):
    info = plsc.get_sparse_core_info()
    NC, NS, L = info.num_cores, info.num_subcores, info.num_lanes
    NW = NC * NS
    assert B % NW == 0
    b_per_w = B // NW
    n_chunks = 4
    chunk = b_per_w // n_chunks
    W = 2 * D
    V4 = V // 4

    mesh = plsc.VectorSubcoreMesh(core_axis_name="c", subcore_axis_name="s")

    @functools.partial(
        pl.kernel,
        mesh=mesh,
        out_type=jax.ShapeDtypeStruct((B,), jnp.float32),
        compiler_params=pltpu.CompilerParams(needs_layout_passes=False),
        scratch_types=[
            pltpu.VMEM((n_chunks, chunk), jnp.int32),
            pltpu.VMEM((n_chunks, chunk), jnp.int32),
            pltpu.VMEM((n_chunks, chunk), jnp.int32),
            pltpu.VMEM((n_chunks, chunk), jnp.int32),
            pltpu.VMEM((chunk, W), jnp.int32),
            pltpu.VMEM((chunk, W), jnp.int32),
            pltpu.VMEM((chunk, W), jnp.int32),
            pltpu.VMEM((chunk, W), jnp.int32),
            pltpu.VMEM((b_per_w,), jnp.float32),
            pltpu.SemaphoreType.DMA,
            pltpu.SemaphoreType.DMA,
            pltpu.SemaphoreType.DMA,
            pltpu.SemaphoreType.DMA,
        ],
    )
    def k(tgt_hbm, ctx_hbm, ttab_hbm, ctab_hbm, out_hbm,
          idx_t, idx_c, sidx_t, sidx_c, rt0, rt1, rc0, rc1, out_v,
          sem_t0, sem_t1, sem_c0, sem_c1):
        wid = lax.axis_index("s") * NC + lax.axis_index("c")
        base = wid * b_per_w
        rows_t = (rt0, rt1)
        rows_c = (rc0, rc1)
        sems_t = (sem_t0, sem_t1)
        sems_c = (sem_c0, sem_c1)

        for j in range(n_chunks):
            pltpu.sync_copy(tgt_hbm.at[pl.ds(base + j * chunk, chunk)], idx_t.at[j])
            pltpu.sync_copy(ctx_hbm.at[pl.ds(base + j * chunk, chunk)], idx_c.at[j])

        for j in range(n_chunks):
            for g in range(chunk // L):
                sl = pl.ds(g * L, L)
                for src, dst in ((idx_t, sidx_t), (idx_c, sidx_c)):
                    v = src[j, sl]
                    dst[j, sl] = jnp.bitwise_or(
                        lax.shift_left(
                            lax.shift_right_logical(v, _LOG_VB), _LOG_QB),
                        jnp.bitwise_and(v, _QB - 1))

        def fire(j):
            b = j % 2
            ct = pltpu.async_copy(ttab_hbm.at[sidx_t.at[j]], rows_t[b], sems_t[b])
            cc = pltpu.async_copy(ctab_hbm.at[sidx_c.at[j]], rows_c[b], sems_c[b])
            return ct, cc

        lanes = lax.iota(jnp.int32, L)
        inflight = [fire(0), fire(1)]

        for j in range(n_chunks):
            b = j % 2
            ct, cc = inflight[b]
            ct.wait()
            cc.wait()
            rt = rows_t[b]
            rc = rows_c[b]

            def group_body(g, carry, j=j, rt=rt, rc=rc):
                sl = pl.ds(g * L, L)
                row = g * L + lanes
                iv_t = idx_t[j, sl]
                iv_c = idx_c[j, sl]
                pt = jnp.bitwise_and(
                    lax.shift_right_logical(iv_t, _LOG_QB), 1) * D
                pc = jnp.bitwise_and(
                    lax.shift_right_logical(iv_c, _LOG_QB), 1) * D
                sh_t = (1 - jnp.bitwise_and(
                    lax.shift_right_logical(iv_t, _LOG_VB - 1), 1)) * 16
                sh_c = (1 - jnp.bitwise_and(
                    lax.shift_right_logical(iv_c, _LOG_VB - 1), 1)) * 16
                hi_mask = jnp.full((L,), -65536, jnp.int32)
                acc = jnp.zeros((L,), jnp.float32)
                for w in range(D):
                    diag = jnp.bitwise_and(lanes + w, D - 1)
                    tw = plsc.load_gather(rt, [row, pt + diag])
                    cw = plsc.load_gather(rc, [row, pc + diag])
                    tf = plsc.bitcast(
                        jnp.bitwise_and(lax.shift_left(tw, sh_t), hi_mask),
                        jnp.float32)
                    cf = plsc.bitcast(
                        jnp.bitwise_and(lax.shift_left(cw, sh_c), hi_mask),
                        jnp.float32)
                    acc = acc + tf * cf
                out_v[pl.ds(j * chunk + g * L, L)] = acc
                return carry

            lax.fori_loop(0, chunk // L, group_body, 0)

            if j + 2 < n_chunks:
                inflight[b] = fire(j + 2)

        pltpu.sync_copy(out_v, out_hbm.at[pl.ds(base, b_per_w)])

    return k


def kernel(target, context, target_table, context_table):
    B = target.shape[0]
    V, D = target_table.shape
    pack = _tc_pack(V, D)
    ttab = pack(target_table.T)
    ctab = pack(context_table.T)
    k = _sc_dot_lookup(B, V, D)
    return k(target.astype(jnp.int32), context.astype(jnp.int32), ttab, ctab)


# speedup vs baseline: 148.9240x; 1.1234x over previous
import functools

import jax
import jax.numpy as jnp
from jax import lax
from jax.experimental import pallas as pl
from jax.experimental.pallas import tpu as pltpu
from jax.experimental.pallas import tpu_sc as plsc


_VB = 32768
_HB = _VB // 2
_QB = _VB // 4
_LOG_VB = 15
_LOG_QB = 13


def _pack_block(xT_ref, out_ref):
    D = xT_ref.shape[0]
    bits = lax.bitcast_convert_type(xT_ref[...], jnp.uint32)
    half = jnp.uint32(0x8000)
    lo = lax.shift_right_logical(
        lax.slice(bits, (0, 0), (D, _HB)) + half, jnp.uint32(16))
    hi = jnp.bitwise_and(
        lax.slice(bits, (0, _HB), (D, _VB)) + half, jnp.uint32(0xFFFF0000))
    pa = jnp.bitwise_or(lo, hi)
    t = lax.transpose(pa, (1, 0))
    h0 = lax.slice(t, (0, 0), (_QB, D))
    h1 = lax.slice(t, (_QB, 0), (_HB, D))
    out_ref[...] = lax.bitcast_convert_type(
        jnp.concatenate([h0, h1], axis=1), jnp.int32)


def _tc_pack(V, D):
    grid = pl.cdiv(V, _VB)
    return pl.pallas_call(
        _pack_block,
        grid=(grid,),
        in_specs=[pl.BlockSpec((D, _VB), lambda b: (0, b))],
        out_specs=pl.BlockSpec((_QB, 2 * D), lambda b: (b, 0)),
        out_shape=jax.ShapeDtypeStruct((grid * _QB, 2 * D), jnp.int32),
    )


def _sc_dot_lookup(B, V, D):
    info = plsc.get_sparse_core_info()
    NC, NS, L = info.num_cores, info.num_subcores, info.num_lanes
    NW = NC * NS
    assert B % NW == 0
    b_per_w = B // NW
    n_chunks = 4
    chunk = b_per_w // n_chunks
    W = 2 * D
    V4 = V // 4

    mesh = plsc.VectorSubcoreMesh(core_axis_name="c", subcore_axis_name="s")

    @functools.partial(
        pl.kernel,
        mesh=mesh,
        out_type=jax.ShapeDtypeStruct((B,), jnp.float32),
        compiler_params=pltpu.CompilerParams(needs_layout_passes=False),
        scratch_types=[
            pltpu.VMEM((n_chunks, chunk), jnp.int32),
            pltpu.VMEM((n_chunks, chunk), jnp.int32),
            pltpu.VMEM((n_chunks, chunk), jnp.int32),
            pltpu.VMEM((n_chunks, chunk), jnp.int32),
            pltpu.VMEM((chunk, W), jnp.int32),
            pltpu.VMEM((chunk, W), jnp.int32),
            pltpu.VMEM((chunk, W), jnp.int32),
            pltpu.VMEM((chunk, W), jnp.int32),
            pltpu.VMEM((b_per_w,), jnp.float32),
            pltpu.SemaphoreType.DMA,
            pltpu.SemaphoreType.DMA,
            pltpu.SemaphoreType.DMA,
            pltpu.SemaphoreType.DMA,
        ],
    )
    def k(tgt_hbm, ctx_hbm, ttab_hbm, ctab_hbm, out_hbm,
          idx_t, idx_c, sidx_t, sidx_c, rt0, rt1, rc0, rc1, out_v,
          sem_t0, sem_t1, sem_c0, sem_c1):
        wid = lax.axis_index("s") * NC + lax.axis_index("c")
        base = wid * b_per_w
        rows_t = (rt0, rt1)
        rows_c = (rc0, rc1)
        sems_t = (sem_t0, sem_t1)
        sems_c = (sem_c0, sem_c1)

        for j in range(n_chunks):
            pltpu.sync_copy(tgt_hbm.at[pl.ds(base + j * chunk, chunk)], idx_t.at[j])
            pltpu.sync_copy(ctx_hbm.at[pl.ds(base + j * chunk, chunk)], idx_c.at[j])

        for j in range(n_chunks):
            for g in range(chunk // L):
                sl = pl.ds(g * L, L)
                for src, dst in ((idx_t, sidx_t), (idx_c, sidx_c)):
                    v = src[j, sl]
                    dst[j, sl] = jnp.bitwise_or(
                        lax.shift_left(
                            lax.shift_right_logical(v, _LOG_VB), _LOG_QB),
                        jnp.bitwise_and(v, _QB - 1))

        def fire(j):
            b = j % 2
            ct = pltpu.async_copy(ttab_hbm.at[sidx_t.at[j]], rows_t[b], sems_t[b])
            cc = pltpu.async_copy(ctab_hbm.at[sidx_c.at[j]], rows_c[b], sems_c[b])
            return ct, cc

        lanes = lax.iota(jnp.int32, L)
        inflight = [fire(0), fire(1)]

        for j in range(n_chunks):
            b = j % 2
            ct, cc = inflight[b]
            ct.wait()
            cc.wait()
            rt = rows_t[b]
            rc = rows_c[b]

            def group_body(g, carry, j=j, rt=rt, rc=rc):
                sl = pl.ds(g * L, L)
                row = g * L + lanes
                iv_t = idx_t[j, sl]
                iv_c = idx_c[j, sl]
                pt = jnp.bitwise_and(
                    lax.shift_right_logical(iv_t, _LOG_QB), 1) * D
                pc = jnp.bitwise_and(
                    lax.shift_right_logical(iv_c, _LOG_QB), 1) * D
                sh_t = (1 - jnp.bitwise_and(
                    lax.shift_right_logical(iv_t, _LOG_VB - 1), 1)) * 16
                sh_c = (1 - jnp.bitwise_and(
                    lax.shift_right_logical(iv_c, _LOG_VB - 1), 1)) * 16
                hi_mask = jnp.full((L,), -65536, jnp.int32)
                acc = jnp.zeros((L,), jnp.float32)
                for w in range(D):
                    diag = jnp.bitwise_and(lanes + w, D - 1)
                    tw = plsc.load_gather(rt, [row, pt + diag])
                    cw = plsc.load_gather(rc, [row, pc + diag])
                    tf = plsc.bitcast(
                        jnp.bitwise_and(lax.shift_left(tw, sh_t), hi_mask),
                        jnp.float32)
                    cf = plsc.bitcast(
                        jnp.bitwise_and(lax.shift_left(cw, sh_c), hi_mask),
                        jnp.float32)
                    acc = acc + tf * cf
                out_v[pl.ds(j * chunk + g * L, L)] = acc
                return carry

            lax.fori_loop(0, chunk // L, group_body, 0)

            if j + 2 < n_chunks:
                inflight[b] = fire(j + 2)

        pltpu.sync_copy(out_v, out_hbm.at[pl.ds(base, b_per_w)])

    return k


def kernel(target, context, target_table, context_table):
    B = target.shape[0]
    V, D = target_table.shape
    pack = _tc_pack(V, D)
    ttab = pack(target_table.T)
    ctab = pack(context_table.T)
    k = _sc_dot_lookup(B, V, D)
    return k(target.astype(jnp.int32), context.astype(jnp.int32), ttab, ctab)
